# baseline (speedup 1.0000x reference)
import functools
import jax
import jax.numpy as jnp
from jax import lax
from jax.experimental import pallas as pl
from jax.experimental.pallas import tpu as pltpu
from jax.experimental.pallas import tpu_sc as plsc

B = 4
N = 20736
WPB = 8
CHUNK = N // WPB
STEPS = CHUNK // 16
NPICK = 300
PPAD = 304
PBLK = PPAD // 16
TOPK = 32
NGT = 10
IOU_THR = 0.5

_f32 = jnp.float32
_i32 = jnp.int32

Y1O, X1O, Y2O, X2O, ARO = (k * CHUNK for k in range(5))


def _kernel_body(anch_hbm, delt_hbm, lab_hbm, gt_hbm, roi_hbm, gti_hbm,
                 av, dv, bd, sc, cand, stage8, gtv,
                 picks, merged, gtid, roi_st, gti_st, shared):
    c = lax.axis_index("c")
    s = lax.axis_index("s")
    b = 2 * c + s // WPB
    part = s % WPB
    gbase = (s // WPB) * WPB
    is_leader = part == 0
    lane = lax.iota(_i32, 16)

    def _bcast(v, j):
        return v[jnp.full((16,), j, _i32)]

    def _hmax(v):
        for off in (8, 4, 2, 1):
            v = jnp.maximum(v, v[lane ^ off])
        return v

    def _hmin(v):
        for off in (8, 4, 2, 1):
            v = jnp.minimum(v, v[lane ^ off])
        return v

    for k in range(4):
        pltpu.sync_copy(
            anch_hbm.at[pl.ds((b * 4 + k) * N + part * CHUNK, CHUNK)],
            av.at[k])
        pltpu.sync_copy(
            delt_hbm.at[pl.ds((b * 4 + k) * N + part * CHUNK, CHUNK)],
            dv.at[k])
    pltpu.sync_copy(lab_hbm.at[pl.ds(b * N + part * CHUNK, CHUNK)], sc)

    @pl.when(is_leader)
    def _():
        pltpu.sync_copy(gt_hbm.at[pl.ds(b * 64, 64)], gtv)

    def dec(t, _):
        sl = pl.ds(t * 16, 16)
        a0 = av[0, sl]; a1 = av[1, sl]; a2 = av[2, sl]; a3 = av[3, sl]
        d0 = dv[0, sl]; d1 = dv[1, sl]; d2 = dv[2, sl]; d3 = dv[3, sl]
        aw = a3 - a1
        ah = a2 - a0
        acx = a1 + 0.5 * aw
        acy = a0 + 0.5 * ah
        bw = jnp.exp(d3) * aw
        bh = jnp.exp(d2) * ah
        bcx = d1 * aw + acx
        bcy = d0 * ah + acy
        y1 = bcy - 0.5 * bh
        x1 = bcx - 0.5 * bw
        y2 = bh + y1
        x2 = bw + x1
        bd[pl.ds(Y1O + t * 16, 16)] = y1
        bd[pl.ds(X1O + t * 16, 16)] = x1
        bd[pl.ds(Y2O + t * 16, 16)] = y2
        bd[pl.ds(X2O + t * 16, 16)] = x2
        bd[pl.ds(ARO + t * 16, 16)] = (
            jnp.maximum(y2 - y1, 0.0) * jnp.maximum(x2 - x1, 0.0))
        return 0

    lax.fori_loop(0, STEPS, dec, 0, unroll=4)

    carry0 = (jnp.full((16,), -jnp.inf, _f32), jnp.zeros((16,), _i32))

    def am0(t, carry):
        mv, mj = carry
        v = sc[pl.ds(t * 16, 16)]
        jv = t * 16 + lane
        upd = v > mv
        return jnp.where(upd, v, mv), jnp.where(upd, jv, mj)

    mv, mj = lax.fori_loop(0, STEPS, am0, carry0, unroll=4)

    def pick(i, carry):
        mv, mj = carry
        m = _hmax(mv)
        jloc = _hmin(jnp.where(mv == m, mj, _i32(1 << 30)))
        bvals = plsc.load_gather(
            bd, [jloc + CHUNK * jnp.clip(lane - 2, 0, 4)])
        v = jnp.where(lane == 0, m, bvals)
        v = jnp.where(lane == 1, (part * CHUNK + jloc).astype(_f32), v)
        cand[...] = v
        p = i % 2
        pltpu.sync_copy(cand, shared.at[pl.ds(p * 256 + s * 16, 16)])
        plsc.subcore_barrier()
        pltpu.sync_copy(shared.at[pl.ds(p * 256 + gbase * 16, WPB * 16)],
                        stage8)

        best = stage8[pl.ds(0, 16)]
        bb = _bcast(best, 0)
        for r in range(1, WPB):
            row = stage8[pl.ds(16 * r, 16)]
            rb = _bcast(row, 0)
            win = rb > bb
            best = jnp.where(win, row, best)
            bb = jnp.where(win, rb, bb)
        wg = _bcast(best, 1).astype(_i32)
        valid = bb > 0.0
        wy1 = jnp.where(valid, _bcast(best, 2), 2.0)
        wx1 = jnp.where(valid, _bcast(best, 3), 2.0)
        wy2 = jnp.where(valid, _bcast(best, 4), -2.0)
        wx2 = jnp.where(valid, _bcast(best, 5), -2.0)
        wa = _bcast(best, 6)

        @pl.when(is_leader)
        def _():
            px = jnp.where(valid, best, jnp.zeros((16,), _f32))
            pidx = i + PPAD * jnp.clip(lane - 2, 0, 3)
            plsc.store_scatter(picks, [pidx], px,
                               mask=(lane >= 2) & (lane < 6))

        ownerv = (wg >= part * CHUNK) & (wg < (part + 1) * CHUNK)
        plsc.store_scatter(sc, [wg - part * CHUNK],
                           jnp.full((16,), -1.0, _f32),
                           mask=ownerv & (lane == 0))

        def fs(t, carry):
            mv, mj = carry
            y1c = bd[pl.ds(Y1O + t * 16, 16)]
            x1c = bd[pl.ds(X1O + t * 16, 16)]
            y2c = bd[pl.ds(Y2O + t * 16, 16)]
            x2c = bd[pl.ds(X2O + t * 16, 16)]
            ac = bd[pl.ds(ARO + t * 16, 16)]
            s0 = sc[pl.ds(t * 16, 16)]
            yy1 = jnp.maximum(wy1, y1c)
            xx1 = jnp.maximum(wx1, x1c)
            yy2 = jnp.minimum(wy2, y2c)
            xx2 = jnp.minimum(wx2, x2c)
            inter = jnp.maximum(yy2 - yy1, 0.0) * jnp.maximum(xx2 - xx1, 0.0)
            supp = inter > IOU_THR * jnp.maximum(ac + wa - inter, 1e-8)
            s1 = jnp.where(supp, -1.0, s0)
            sc[pl.ds(t * 16, 16)] = s1
            jv = t * 16 + lane
            upd = s1 > mv
            return jnp.where(upd, s1, mv), jnp.where(upd, jv, mj)

        return lax.fori_loop(0, STEPS, fs, carry0, unroll=6)

    lax.fori_loop(0, NPICK, pick, (mv, mj))

    @pl.when(is_leader)
    def _():
        gr_y1 = gtv[pl.ds(0, 16)]
        gr_x1 = gtv[pl.ds(16, 16)]
        gr_y2 = gtv[pl.ds(32, 16)]
        gr_x2 = gtv[pl.ds(48, 16)]

        def pc(tb, _):
            sl = pl.ds(tb * 16, 16)
            p0 = jnp.clip(picks[pl.ds(0 * PPAD + tb * 16, 16)], 0.0, 1.0)
            p1 = jnp.clip(picks[pl.ds(1 * PPAD + tb * 16, 16)], 0.0, 1.0)
            p2 = jnp.clip(picks[pl.ds(2 * PPAD + tb * 16, 16)], 0.0, 1.0)
            p3 = jnp.clip(picks[pl.ds(3 * PPAD + tb * 16, 16)], 0.0, 1.0)
            pa = jnp.maximum(p2 - p0, 0.0) * jnp.maximum(p3 - p1, 0.0)
            mg = jnp.full((16,), -1.0, _f32)
            gi = jnp.zeros((16,), _i32)
            for g in range(NGT):
                gy1 = _bcast(gr_y1, g); gx1 = _bcast(gr_x1, g)
                gy2 = _bcast(gr_y2, g); gx2 = _bcast(gr_x2, g)
                gar = (jnp.maximum(gy2 - gy1, 0.0)
                       * jnp.maximum(gx2 - gx1, 0.0))
                yy1 = jnp.maximum(p0, gy1)
                xx1 = jnp.maximum(p1, gx1)
                yy2 = jnp.minimum(p2, gy2)
                xx2 = jnp.minimum(p3, gx2)
                inter = (jnp.maximum(yy2 - yy1, 0.0)
                         * jnp.maximum(xx2 - xx1, 0.0))
                iou = inter / jnp.maximum(pa + gar - inter, 1e-8)
                upd = iou > mg
                mg = jnp.where(upd, iou, mg)
                gi = jnp.where(upd, _i32(g), gi)
            jv = tb * 16 + lane
            mg = jnp.where(jv < NPICK, mg, -2.0)
            merged[sl] = mg
            gtid[sl] = gi
            return 0

        lax.fori_loop(0, PBLK, pc, 0)

        def ext(k, _):
            def am(t, carry):
                mv, mj = carry
                v = merged[pl.ds(t * 16, 16)]
                jv = t * 16 + lane
                upd = v > mv
                return jnp.where(upd, v, mv), jnp.where(upd, jv, mj)

            mv, mj = lax.fori_loop(0, PBLK, am, carry0)
            m = _hmax(mv)
            jsel = _hmin(jnp.where(mv == m, mj, _i32(1 << 30)))
            coords = plsc.load_gather(
                picks, [jsel + PPAD * jnp.minimum(lane, 3)])
            coords = jnp.clip(coords, 0.0, 1.0)
            plsc.store_scatter(roi_st, [4 * k + lane], coords, mask=lane < 4)
            gsel = plsc.load_gather(gtid, [jsel])
            plsc.store_scatter(gti_st, [jnp.zeros((16,), _i32) + k], gsel,
                               mask=lane == 0)
            plsc.store_scatter(merged, [jsel],
                               jnp.full((16,), -3.0, _f32), mask=lane == 0)
            return 0

        lax.fori_loop(0, TOPK, ext, 0)
        pltpu.sync_copy(roi_st, roi_hbm.at[pl.ds(b * TOPK * 4, TOPK * 4)])
        pltpu.sync_copy(gti_st, gti_hbm.at[pl.ds(b * TOPK, TOPK)])


_nms_call = pl.kernel(
    _kernel_body,
    out_type=(jax.ShapeDtypeStruct((B * TOPK * 4,), _f32),
              jax.ShapeDtypeStruct((B * TOPK,), _i32)),
    mesh=plsc.VectorSubcoreMesh(core_axis_name="c", subcore_axis_name="s"),
    compiler_params=pltpu.CompilerParams(needs_layout_passes=False, use_tc_tiling_on_sc=False),
    scratch_types=[
        pltpu.VMEM((4, CHUNK), _f32),
        pltpu.VMEM((4, CHUNK), _f32),
        pltpu.VMEM((5 * CHUNK,), _f32),
        pltpu.VMEM((CHUNK,), _f32),
        pltpu.VMEM((16,), _f32),
        pltpu.VMEM((WPB * 16,), _f32),
        pltpu.VMEM((64,), _f32),
        pltpu.VMEM((4 * PPAD,), _f32),
        pltpu.VMEM((PPAD,), _f32),
        pltpu.VMEM((PPAD,), _i32),
        pltpu.VMEM((TOPK * 4,), _f32),
        pltpu.VMEM((TOPK,), _i32),
        pltpu.VMEM_SHARED((2 * 16 * 16,), _f32),
    ],
)


@jax.jit
def kernel(rpn_bbox_deltas, rpn_labels, anchors, gt_boxes):
    anch_t = anchors.transpose(0, 2, 1).reshape(-1)
    delt_t = rpn_bbox_deltas.reshape(B, N, 4).transpose(0, 2, 1).reshape(-1)
    lab = rpn_labels.reshape(-1)
    gt_t = jnp.pad(gt_boxes.transpose(0, 2, 1),
                   ((0, 0), (0, 0), (0, 16 - NGT))).reshape(-1)
    roi_pos, gt_idx = _nms_call(anch_t, delt_t, lab, gt_t)
    roi = jnp.concatenate(
        [roi_pos.reshape(B, TOPK, 4), jnp.zeros((B, 128 - TOPK, 4), _f32)],
        axis=1)
    return lax.stop_gradient(roi), lax.stop_gradient(gt_idx.reshape(B, TOPK))


# speedup vs baseline: 2.2239x; 2.2239x over previous
import functools
import jax
import jax.numpy as jnp
from jax import lax
from jax.experimental import pallas as pl
from jax.experimental.pallas import tpu as pltpu
from jax.experimental.pallas import tpu_sc as plsc

B = 4
N = 20736
WPB = 8
CHUNK = N // WPB
STEPS = CHUNK // 16
NPICK = 300
PPAD = 304
PBLK = PPAD // 16
TOPK = 32
NGT = 10
IOU_THR = 0.5

_f32 = jnp.float32
_i32 = jnp.int32

Y1O, X1O, Y2O, X2O, ARO = (k * CHUNK for k in range(5))


def _kernel_body(anch_hbm, delt_hbm, lab_hbm, gt_hbm, roi_hbm, gti_hbm,
                 av, dv, bd, sc, cand, stage8, gtv,
                 picks, merged, gtid, roi_st, gti_st, shared):
    c = lax.axis_index("c")
    s = lax.axis_index("s")
    b = 2 * c + s // WPB
    part = s % WPB
    gbase = (s // WPB) * WPB
    is_leader = part == 0
    lane = lax.iota(_i32, 16)

    def _bcast(v, j):
        return v[jnp.full((16,), j, _i32)]

    def _hmax(v):
        for off in (8, 4, 2, 1):
            v = jnp.maximum(v, v[lane ^ off])
        return v

    def _hmin(v):
        for off in (8, 4, 2, 1):
            v = jnp.minimum(v, v[lane ^ off])
        return v

    for k in range(4):
        pltpu.sync_copy(
            anch_hbm.at[pl.ds((b * 4 + k) * N + part * CHUNK, CHUNK)],
            av.at[k])
        pltpu.sync_copy(
            delt_hbm.at[pl.ds((b * 4 + k) * N + part * CHUNK, CHUNK)],
            dv.at[k])
    pltpu.sync_copy(lab_hbm.at[pl.ds(b * N + part * CHUNK, CHUNK)], sc)

    @pl.when(is_leader)
    def _():
        pltpu.sync_copy(gt_hbm.at[pl.ds(b * 64, 64)], gtv)

    def dec(t, _):
        sl = pl.ds(t * 16, 16)
        a0 = av[0, sl]; a1 = av[1, sl]; a2 = av[2, sl]; a3 = av[3, sl]
        d0 = dv[0, sl]; d1 = dv[1, sl]; d2 = dv[2, sl]; d3 = dv[3, sl]
        aw = a3 - a1
        ah = a2 - a0
        acx = a1 + 0.5 * aw
        acy = a0 + 0.5 * ah
        bw = jnp.exp(d3) * aw
        bh = jnp.exp(d2) * ah
        bcx = d1 * aw + acx
        bcy = d0 * ah + acy
        y1 = bcy - 0.5 * bh
        x1 = bcx - 0.5 * bw
        y2 = bh + y1
        x2 = bw + x1
        bd[pl.ds(Y1O + t * 16, 16)] = y1
        bd[pl.ds(X1O + t * 16, 16)] = x1
        bd[pl.ds(Y2O + t * 16, 16)] = y2
        bd[pl.ds(X2O + t * 16, 16)] = x2
        bd[pl.ds(ARO + t * 16, 16)] = (
            jnp.maximum(y2 - y1, 0.0) * jnp.maximum(x2 - x1, 0.0))
        return 0

    plsc.parallel_loop(0, STEPS, 1, unroll=4)(lambda t: dec(t, 0) and None)

    carry0 = (jnp.full((16,), -jnp.inf, _f32), jnp.zeros((16,), _i32))

    def am0(t, carry):
        mv, mj = carry
        v = sc[pl.ds(t * 16, 16)]
        jv = t * 16 + lane
        upd = v > mv
        return jnp.where(upd, v, mv), jnp.where(upd, jv, mj)

    mv, mj = plsc.parallel_loop(0, STEPS, 1, unroll=4, carry=carry0)(am0)

    def pick(i, carry):
        mv, mj = carry
        m = _hmax(mv)
        jloc = _hmin(jnp.where(mv == m, mj, _i32(1 << 30)))
        bvals = plsc.load_gather(
            bd, [jloc + CHUNK * jnp.clip(lane - 2, 0, 4)])
        v = jnp.where(lane == 0, m, bvals)
        v = jnp.where(lane == 1, (part * CHUNK + jloc).astype(_f32), v)
        cand[...] = v
        p = i % 2
        pltpu.sync_copy(cand, shared.at[pl.ds(p * 256 + s * 16, 16)])
        plsc.subcore_barrier()
        pltpu.sync_copy(shared.at[pl.ds(p * 256 + gbase * 16, WPB * 16)],
                        stage8)

        best = stage8[pl.ds(0, 16)]
        bb = _bcast(best, 0)
        for r in range(1, WPB):
            row = stage8[pl.ds(16 * r, 16)]
            rb = _bcast(row, 0)
            win = rb > bb
            best = jnp.where(win, row, best)
            bb = jnp.where(win, rb, bb)
        wg = _bcast(best, 1).astype(_i32)
        valid = bb > 0.0
        wy1 = jnp.where(valid, _bcast(best, 2), 2.0)
        wx1 = jnp.where(valid, _bcast(best, 3), 2.0)
        wy2 = jnp.where(valid, _bcast(best, 4), -2.0)
        wx2 = jnp.where(valid, _bcast(best, 5), -2.0)
        wa = _bcast(best, 6)

        @pl.when(is_leader)
        def _():
            px = jnp.where(valid, best, jnp.zeros((16,), _f32))
            pidx = i + PPAD * jnp.clip(lane - 2, 0, 3)
            plsc.store_scatter(picks, [pidx], px,
                               mask=(lane >= 2) & (lane < 6))

        ownerv = (wg >= part * CHUNK) & (wg < (part + 1) * CHUNK)
        plsc.store_scatter(sc, [wg - part * CHUNK],
                           jnp.full((16,), -1.0, _f32),
                           mask=ownerv & (lane == 0))

        def fs(t, carry):
            mv, mj = carry
            y1c = bd[pl.ds(Y1O + t * 16, 16)]
            x1c = bd[pl.ds(X1O + t * 16, 16)]
            y2c = bd[pl.ds(Y2O + t * 16, 16)]
            x2c = bd[pl.ds(X2O + t * 16, 16)]
            ac = bd[pl.ds(ARO + t * 16, 16)]
            s0 = sc[pl.ds(t * 16, 16)]
            yy1 = jnp.maximum(wy1, y1c)
            xx1 = jnp.maximum(wx1, x1c)
            yy2 = jnp.minimum(wy2, y2c)
            xx2 = jnp.minimum(wx2, x2c)
            inter = jnp.maximum(yy2 - yy1, 0.0) * jnp.maximum(xx2 - xx1, 0.0)
            supp = inter > IOU_THR * jnp.maximum(ac + wa - inter, 1e-8)
            s1 = jnp.where(supp, -1.0, s0)
            sc[pl.ds(t * 16, 16)] = s1
            jv = t * 16 + lane
            upd = s1 > mv
            return jnp.where(upd, s1, mv), jnp.where(upd, jv, mj)

        return plsc.parallel_loop(0, STEPS, 1, unroll=8, carry=carry0)(fs)

    lax.fori_loop(0, NPICK, pick, (mv, mj))

    @pl.when(is_leader)
    def _():
        gr_y1 = gtv[pl.ds(0, 16)]
        gr_x1 = gtv[pl.ds(16, 16)]
        gr_y2 = gtv[pl.ds(32, 16)]
        gr_x2 = gtv[pl.ds(48, 16)]

        def pc(tb, _):
            sl = pl.ds(tb * 16, 16)
            p0 = jnp.clip(picks[pl.ds(0 * PPAD + tb * 16, 16)], 0.0, 1.0)
            p1 = jnp.clip(picks[pl.ds(1 * PPAD + tb * 16, 16)], 0.0, 1.0)
            p2 = jnp.clip(picks[pl.ds(2 * PPAD + tb * 16, 16)], 0.0, 1.0)
            p3 = jnp.clip(picks[pl.ds(3 * PPAD + tb * 16, 16)], 0.0, 1.0)
            pa = jnp.maximum(p2 - p0, 0.0) * jnp.maximum(p3 - p1, 0.0)
            mg = jnp.full((16,), -1.0, _f32)
            gi = jnp.zeros((16,), _i32)
            for g in range(NGT):
                gy1 = _bcast(gr_y1, g); gx1 = _bcast(gr_x1, g)
                gy2 = _bcast(gr_y2, g); gx2 = _bcast(gr_x2, g)
                gar = (jnp.maximum(gy2 - gy1, 0.0)
                       * jnp.maximum(gx2 - gx1, 0.0))
                yy1 = jnp.maximum(p0, gy1)
                xx1 = jnp.maximum(p1, gx1)
                yy2 = jnp.minimum(p2, gy2)
                xx2 = jnp.minimum(p3, gx2)
                inter = (jnp.maximum(yy2 - yy1, 0.0)
                         * jnp.maximum(xx2 - xx1, 0.0))
                iou = inter / jnp.maximum(pa + gar - inter, 1e-8)
                upd = iou > mg
                mg = jnp.where(upd, iou, mg)
                gi = jnp.where(upd, _i32(g), gi)
            jv = tb * 16 + lane
            mg = jnp.where(jv < NPICK, mg, -2.0)
            merged[sl] = mg
            gtid[sl] = gi
            return 0

        lax.fori_loop(0, PBLK, pc, 0)

        def ext(k, _):
            def am(t, carry):
                mv, mj = carry
                v = merged[pl.ds(t * 16, 16)]
                jv = t * 16 + lane
                upd = v > mv
                return jnp.where(upd, v, mv), jnp.where(upd, jv, mj)

            mv, mj = lax.fori_loop(0, PBLK, am, carry0)
            m = _hmax(mv)
            jsel = _hmin(jnp.where(mv == m, mj, _i32(1 << 30)))
            coords = plsc.load_gather(
                picks, [jsel + PPAD * jnp.minimum(lane, 3)])
            coords = jnp.clip(coords, 0.0, 1.0)
            plsc.store_scatter(roi_st, [4 * k + lane], coords, mask=lane < 4)
            gsel = plsc.load_gather(gtid, [jsel])
            plsc.store_scatter(gti_st, [jnp.zeros((16,), _i32) + k], gsel,
                               mask=lane == 0)
            plsc.store_scatter(merged, [jsel],
                               jnp.full((16,), -3.0, _f32), mask=lane == 0)
            return 0

        lax.fori_loop(0, TOPK, ext, 0)
        pltpu.sync_copy(roi_st, roi_hbm.at[pl.ds(b * TOPK * 4, TOPK * 4)])
        pltpu.sync_copy(gti_st, gti_hbm.at[pl.ds(b * TOPK, TOPK)])


_nms_call = pl.kernel(
    _kernel_body,
    out_type=(jax.ShapeDtypeStruct((B * TOPK * 4,), _f32),
              jax.ShapeDtypeStruct((B * TOPK,), _i32)),
    mesh=plsc.VectorSubcoreMesh(core_axis_name="c", subcore_axis_name="s"),
    compiler_params=pltpu.CompilerParams(needs_layout_passes=False, use_tc_tiling_on_sc=False),
    scratch_types=[
        pltpu.VMEM((4, CHUNK), _f32),
        pltpu.VMEM((4, CHUNK), _f32),
        pltpu.VMEM((5 * CHUNK,), _f32),
        pltpu.VMEM((CHUNK,), _f32),
        pltpu.VMEM((16,), _f32),
        pltpu.VMEM((WPB * 16,), _f32),
        pltpu.VMEM((64,), _f32),
        pltpu.VMEM((4 * PPAD,), _f32),
        pltpu.VMEM((PPAD,), _f32),
        pltpu.VMEM((PPAD,), _i32),
        pltpu.VMEM((TOPK * 4,), _f32),
        pltpu.VMEM((TOPK,), _i32),
        pltpu.VMEM_SHARED((2 * 16 * 16,), _f32),
    ],
)


@jax.jit
def kernel(rpn_bbox_deltas, rpn_labels, anchors, gt_boxes):
    anch_t = anchors.transpose(0, 2, 1).reshape(-1)
    delt_t = rpn_bbox_deltas.reshape(B, N, 4).transpose(0, 2, 1).reshape(-1)
    lab = rpn_labels.reshape(-1)
    gt_t = jnp.pad(gt_boxes.transpose(0, 2, 1),
                   ((0, 0), (0, 0), (0, 16 - NGT))).reshape(-1)
    roi_pos, gt_idx = _nms_call(anch_t, delt_t, lab, gt_t)
    roi = jnp.concatenate(
        [roi_pos.reshape(B, TOPK, 4), jnp.zeros((B, 128 - TOPK, 4), _f32)],
        axis=1)
    return lax.stop_gradient(roi), lax.stop_gradient(gt_idx.reshape(B, TOPK))


# speedup vs baseline: 2.2469x; 1.0104x over previous
import functools
import jax
import jax.numpy as jnp
from jax import lax
from jax.experimental import pallas as pl
from jax.experimental.pallas import tpu as pltpu
from jax.experimental.pallas import tpu_sc as plsc

B = 4
N = 20736
WPB = 8
CHUNK = N // WPB
STEPS = CHUNK // 16
NPICK = 300
PPAD = 304
PBLK = PPAD // 16
TOPK = 32
NGT = 10
IOU_THR = 0.5

_f32 = jnp.float32
_i32 = jnp.int32

Y1O, X1O, Y2O, X2O, ARO = (k * CHUNK for k in range(5))


def _kernel_body(anch_hbm, delt_hbm, lab_hbm, gt_hbm, roi_hbm, gti_hbm,
                 av, dv, bd, sc, cand, stage, gtv,
                 picks, merged, gtid, roi_st, gti_st, shared):
    c = lax.axis_index("c")
    s = lax.axis_index("s")
    b = 2 * c + s // WPB
    part = s % WPB
    gbase = (s // WPB) * WPB
    obase = 8 - gbase
    is_leader = part == 0
    lane = lax.iota(_i32, 16)

    def _bcast(v, j):
        return v[jnp.full((16,), j, _i32)]

    def _hmax(v):
        for off in (8, 4, 2, 1):
            v = jnp.maximum(v, v[lane ^ off])
        return v

    def _hmin(v):
        for off in (8, 4, 2, 1):
            v = jnp.minimum(v, v[lane ^ off])
        return v

    def _lexmax(v, j, l):
        for off in (8, 4, 2, 1):
            pv = v[lane ^ off]
            pj = j[lane ^ off]
            pL = l[lane ^ off]
            take = (pv > v) | ((pv == v) & (pj < j))
            v = jnp.where(take, pv, v)
            j = jnp.where(take, pj, j)
            l = jnp.where(take, pL, l)
        return v, j, l

    for k in range(4):
        pltpu.sync_copy(
            anch_hbm.at[pl.ds((b * 4 + k) * N + part * CHUNK, CHUNK)],
            av.at[k])
        pltpu.sync_copy(
            delt_hbm.at[pl.ds((b * 4 + k) * N + part * CHUNK, CHUNK)],
            dv.at[k])
    pltpu.sync_copy(lab_hbm.at[pl.ds(b * N + part * CHUNK, CHUNK)], sc)

    @pl.when(is_leader)
    def _():
        pltpu.sync_copy(gt_hbm.at[pl.ds(b * 64, 64)], gtv)

    def dec(t):
        sl = pl.ds(t * 16, 16)
        a0 = av[0, sl]; a1 = av[1, sl]; a2 = av[2, sl]; a3 = av[3, sl]
        d0 = dv[0, sl]; d1 = dv[1, sl]; d2 = dv[2, sl]; d3 = dv[3, sl]
        aw = a3 - a1
        ah = a2 - a0
        acx = a1 + 0.5 * aw
        acy = a0 + 0.5 * ah
        bw = jnp.exp(d3) * aw
        bh = jnp.exp(d2) * ah
        bcx = d1 * aw + acx
        bcy = d0 * ah + acy
        y1 = bcy - 0.5 * bh
        x1 = bcx - 0.5 * bw
        y2 = bh + y1
        x2 = bw + x1
        bd[pl.ds(Y1O + t * 16, 16)] = y1
        bd[pl.ds(X1O + t * 16, 16)] = x1
        bd[pl.ds(Y2O + t * 16, 16)] = y2
        bd[pl.ds(X2O + t * 16, 16)] = x2
        bd[pl.ds(ARO + t * 16, 16)] = (
            jnp.maximum(y2 - y1, 0.0) * jnp.maximum(x2 - x1, 0.0))

    plsc.parallel_loop(0, STEPS, 1, unroll=4)(dec)

    ninf = jnp.full((16,), -jnp.inf, _f32)
    zero_i = jnp.zeros((16,), _i32)
    top2_0 = (ninf, zero_i, ninf, zero_i)

    def _top2_step(t, carry, s1):
        v1, j1, v2, j2 = carry
        jvv = t * 16 + lane
        u1 = s1 > v1
        u2 = (s1 > v2) & (~u1)
        nv2 = jnp.where(u1, v1, jnp.where(u2, s1, v2))
        nj2 = jnp.where(u1, j1, jnp.where(u2, jvv, j2))
        nv1 = jnp.where(u1, s1, v1)
        nj1 = jnp.where(u1, jvv, j1)
        return nv1, nj1, nv2, nj2

    def am0(t, carry):
        return _top2_step(t, carry, sc[pl.ds(t * 16, 16)])

    top2 = plsc.parallel_loop(0, STEPS, 1, unroll=4, carry=top2_0)(am0)

    def cond(carry):
        return carry[2] == 1

    def round_body(carry):
        i, r, _, v1, j1, v2, j2 = carry
        actives = i < NPICK
        activev = zero_i + i < NPICK

        c1v, c1j, _ = _lexmax(v1, j1, lane)
        winm = (v1 == c1v) & (j1 == c1j)
        c2v, c2j, _ = _lexmax(jnp.where(winm, v2, v1),
                              jnp.where(winm, j2, j1), lane)

        jboth = jnp.where(lane < 8, c1j, c2j)
        row = plsc.load_gather(
            bd, [jboth + CHUNK * jnp.clip((lane & 7) - 2, 0, 4)])
        row = jnp.where(lane == 0, c1v, row)
        row = jnp.where(lane == 1, (part * CHUNK + c1j).astype(_f32), row)
        row = jnp.where(lane == 7, i.astype(_f32), row)
        row = jnp.where(lane == 8, c2v, row)
        row = jnp.where(lane == 9, (part * CHUNK + c2j).astype(_f32), row)
        cand[...] = row
        p = r % 2
        pltpu.sync_copy(cand, shared.at[pl.ds(p * 256 + s * 16, 16)])
        plsc.subcore_barrier()
        pltpu.sync_copy(shared.at[pl.ds(p * 256, 256)], stage)

        base = gbase * 16 + (lane & 7) * 16 + (lane >> 3) * 8
        sv = plsc.load_gather(stage, [base])
        jv = plsc.load_gather(stage, [base + 1]).astype(_i32)
        y1v = plsc.load_gather(stage, [base + 2])
        x1v = plsc.load_gather(stage, [base + 3])
        y2v = plsc.load_gather(stage, [base + 4])
        x2v = plsc.load_gather(stage, [base + 5])
        avv = plsc.load_gather(stage, [base + 6])

        w1v, w1j, w1l = _lexmax(sv, jv, lane)
        w1y1 = y1v[w1l]; w1x1 = x1v[w1l]
        w1y2 = y2v[w1l]; w1x2 = x2v[w1l]
        w1a = avv[w1l]
        valid1 = (w1v > 0.0) & activev
        w1y1d = jnp.where(valid1, w1y1, 2.0)
        w1x1d = jnp.where(valid1, w1x1, 2.0)
        w1y2d = jnp.where(valid1, w1y2, -2.0)
        w1x2d = jnp.where(valid1, w1x2, -2.0)

        yy1 = jnp.maximum(w1y1d, y1v)
        xx1 = jnp.maximum(w1x1d, x1v)
        yy2 = jnp.minimum(w1y2d, y2v)
        xx2 = jnp.minimum(w1x2d, x2v)
        inter = jnp.maximum(yy2 - yy1, 0.0) * jnp.maximum(xx2 - xx1, 0.0)
        dead = (inter > IOU_THR * jnp.maximum(avv + w1a - inter, 1e-8)) \
            | (jv == w1j)

        w2v, w2j, w2l = _lexmax(jnp.where(dead, -jnp.inf, sv), jv, lane)
        w2y1 = y1v[w2l]; w2x1 = x1v[w2l]
        w2y2 = y2v[w2l]; w2x2 = x2v[w2l]
        w2a = avv[w2l]
        di = jnp.where(dead, 1, 0)
        pairdead = _hmax(di & di[lane ^ 8])
        ok2v = (pairdead == 0) & (zero_i + i + 1 < NPICK) & activev
        ok2s = jnp.where(ok2v, 1, 0)[0] == 1
        valid2 = (w2v > 0.0) & ok2v
        w2y1d = jnp.where(valid2, w2y1, 2.0)
        w2x1d = jnp.where(valid2, w2x1, 2.0)
        w2y2d = jnp.where(valid2, w2y2, -2.0)
        w2x2d = jnp.where(valid2, w2x2, -2.0)

        neg1 = jnp.full((16,), -1.0, _f32)
        own1 = (w1j >= part * CHUNK) & (w1j < (part + 1) * CHUNK) & activev
        plsc.store_scatter(sc, [jnp.clip(w1j - part * CHUNK, 0, CHUNK - 1)],
                           neg1, mask=own1 & (lane == 0))
        own2 = (w2j >= part * CHUNK) & (w2j < (part + 1) * CHUNK) & ok2v
        plsc.store_scatter(sc, [jnp.clip(w2j - part * CHUNK, 0, CHUNK - 1)],
                           neg1, mask=own2 & (lane == 0))

        @pl.when(is_leader)
        def _():
            px1 = jnp.where(lane == 2, w1y1, 0.0)
            px1 = jnp.where(lane == 3, w1x1, px1)
            px1 = jnp.where(lane == 4, w1y2, px1)
            px1 = jnp.where(lane == 5, w1x2, px1)
            px1 = jnp.where(valid1, px1, 0.0)
            pidx = PPAD * jnp.clip(lane - 2, 0, 3)
            lmask = (lane >= 2) & (lane < 6)
            plsc.store_scatter(picks, [i + pidx], px1,
                               mask=lmask & activev)
            px2 = jnp.where(lane == 2, w2y1, 0.0)
            px2 = jnp.where(lane == 3, w2x1, px2)
            px2 = jnp.where(lane == 4, w2y2, px2)
            px2 = jnp.where(lane == 5, w2x2, px2)
            px2 = jnp.where(valid2, px2, 0.0)
            plsc.store_scatter(picks, [i + 1 + pidx], px2,
                               mask=lmask & ok2v)

        def fs(t, carry):
            y1c = bd[pl.ds(Y1O + t * 16, 16)]
            x1c = bd[pl.ds(X1O + t * 16, 16)]
            y2c = bd[pl.ds(Y2O + t * 16, 16)]
            x2c = bd[pl.ds(X2O + t * 16, 16)]
            ac = bd[pl.ds(ARO + t * 16, 16)]
            s0 = sc[pl.ds(t * 16, 16)]
            i1 = (jnp.maximum(jnp.minimum(w1y2d, y2c)
                              - jnp.maximum(w1y1d, y1c), 0.0)
                  * jnp.maximum(jnp.minimum(w1x2d, x2c)
                                - jnp.maximum(w1x1d, x1c), 0.0))
            i2 = (jnp.maximum(jnp.minimum(w2y2d, y2c)
                              - jnp.maximum(w2y1d, y1c), 0.0)
                  * jnp.maximum(jnp.minimum(w2x2d, x2c)
                                - jnp.maximum(w2x1d, x1c), 0.0))
            supp = (i1 > IOU_THR * jnp.maximum(ac + w1a - i1, 1e-8)) \
                | (i2 > IOU_THR * jnp.maximum(ac + w2a - i2, 1e-8))
            s1 = jnp.where(supp, -1.0, s0)
            sc[pl.ds(t * 16, 16)] = s1
            return _top2_step(t, carry, s1)

        nt = plsc.parallel_loop(0, STEPS, 1, unroll=8, carry=top2_0)(fs)

        orow = stage[pl.ds(obase * 16, 16)]
        conts = (i < NPICK) | (orow[7] < float(NPICK))
        ni = jnp.where(actives, i + 1 + jnp.where(ok2s, 1, 0), i)
        return (ni, r + 1, jnp.where(conts, _i32(1), _i32(0)),
                nt[0], nt[1], nt[2], nt[3])

    lax.while_loop(cond, round_body,
                   (_i32(0), _i32(0), _i32(1),
                    top2[0], top2[1], top2[2], top2[3]))

    carry0 = (ninf, zero_i)

    @pl.when(is_leader)
    def _():
        gr_y1 = gtv[pl.ds(0, 16)]
        gr_x1 = gtv[pl.ds(16, 16)]
        gr_y2 = gtv[pl.ds(32, 16)]
        gr_x2 = gtv[pl.ds(48, 16)]

        def pc(tb, _):
            sl = pl.ds(tb * 16, 16)
            p0 = jnp.clip(picks[pl.ds(0 * PPAD + tb * 16, 16)], 0.0, 1.0)
            p1 = jnp.clip(picks[pl.ds(1 * PPAD + tb * 16, 16)], 0.0, 1.0)
            p2 = jnp.clip(picks[pl.ds(2 * PPAD + tb * 16, 16)], 0.0, 1.0)
            p3 = jnp.clip(picks[pl.ds(3 * PPAD + tb * 16, 16)], 0.0, 1.0)
            pa = jnp.maximum(p2 - p0, 0.0) * jnp.maximum(p3 - p1, 0.0)
            mg = jnp.full((16,), -1.0, _f32)
            gi = jnp.zeros((16,), _i32)
            for g in range(NGT):
                gy1 = _bcast(gr_y1, g); gx1 = _bcast(gr_x1, g)
                gy2 = _bcast(gr_y2, g); gx2 = _bcast(gr_x2, g)
                gar = (jnp.maximum(gy2 - gy1, 0.0)
                       * jnp.maximum(gx2 - gx1, 0.0))
                yy1 = jnp.maximum(p0, gy1)
                xx1 = jnp.maximum(p1, gx1)
                yy2 = jnp.minimum(p2, gy2)
                xx2 = jnp.minimum(p3, gx2)
                inter = (jnp.maximum(yy2 - yy1, 0.0)
                         * jnp.maximum(xx2 - xx1, 0.0))
                iou = inter / jnp.maximum(pa + gar - inter, 1e-8)
                upd = iou > mg
                mg = jnp.where(upd, iou, mg)
                gi = jnp.where(upd, _i32(g), gi)
            jv = tb * 16 + lane
            mg = jnp.where(jv < NPICK, mg, -2.0)
            merged[sl] = mg
            gtid[sl] = gi
            return 0

        lax.fori_loop(0, PBLK, pc, 0)

        def ext(k, _):
            def am(t, carry):
                mv, mj = carry
                v = merged[pl.ds(t * 16, 16)]
                jv = t * 16 + lane
                upd = v > mv
                return jnp.where(upd, v, mv), jnp.where(upd, jv, mj)

            mv, mj = lax.fori_loop(0, PBLK, am, carry0)
            m = _hmax(mv)
            jsel = _hmin(jnp.where(mv == m, mj, _i32(1 << 30)))
            coords = plsc.load_gather(
                picks, [jsel + PPAD * jnp.minimum(lane, 3)])
            coords = jnp.clip(coords, 0.0, 1.0)
            plsc.store_scatter(roi_st, [4 * k + lane], coords, mask=lane < 4)
            gsel = plsc.load_gather(gtid, [jsel])
            plsc.store_scatter(gti_st, [jnp.zeros((16,), _i32) + k], gsel,
                               mask=lane == 0)
            plsc.store_scatter(merged, [jsel],
                               jnp.full((16,), -3.0, _f32), mask=lane == 0)
            return 0

        lax.fori_loop(0, TOPK, ext, 0)
        pltpu.sync_copy(roi_st, roi_hbm.at[pl.ds(b * TOPK * 4, TOPK * 4)])
        pltpu.sync_copy(gti_st, gti_hbm.at[pl.ds(b * TOPK, TOPK)])


_nms_call = pl.kernel(
    _kernel_body,
    out_type=(jax.ShapeDtypeStruct((B * TOPK * 4,), _f32),
              jax.ShapeDtypeStruct((B * TOPK,), _i32)),
    mesh=plsc.VectorSubcoreMesh(core_axis_name="c", subcore_axis_name="s"),
    compiler_params=pltpu.CompilerParams(needs_layout_passes=False,
                                         use_tc_tiling_on_sc=False),
    scratch_types=[
        pltpu.VMEM((4, CHUNK), _f32),
        pltpu.VMEM((4, CHUNK), _f32),
        pltpu.VMEM((5 * CHUNK,), _f32),
        pltpu.VMEM((CHUNK,), _f32),
        pltpu.VMEM((16,), _f32),
        pltpu.VMEM((256,), _f32),
        pltpu.VMEM((64,), _f32),
        pltpu.VMEM((4 * PPAD,), _f32),
        pltpu.VMEM((PPAD,), _f32),
        pltpu.VMEM((PPAD,), _i32),
        pltpu.VMEM((TOPK * 4,), _f32),
        pltpu.VMEM((TOPK,), _i32),
        pltpu.VMEM_SHARED((2 * 16 * 16,), _f32),
    ],
)


@jax.jit
def kernel(rpn_bbox_deltas, rpn_labels, anchors, gt_boxes):
    anch_t = anchors.transpose(0, 2, 1).reshape(-1)
    delt_t = rpn_bbox_deltas.reshape(B, N, 4).transpose(0, 2, 1).reshape(-1)
    lab = rpn_labels.reshape(-1)
    gt_t = jnp.pad(gt_boxes.transpose(0, 2, 1),
                   ((0, 0), (0, 0), (0, 16 - NGT))).reshape(-1)
    roi_pos, gt_idx = _nms_call(anch_t, delt_t, lab, gt_t)
    roi = jnp.concatenate(
        [roi_pos.reshape(B, TOPK, 4), jnp.zeros((B, 128 - TOPK, 4), _f32)],
        axis=1)
    return lax.stop_gradient(roi), lax.stop_gradient(gt_idx.reshape(B, TOPK))


# speedup vs baseline: 2.2916x; 1.0199x over previous
import functools
import jax
import jax.numpy as jnp
from jax import lax
from jax.experimental import pallas as pl
from jax.experimental.pallas import tpu as pltpu
from jax.experimental.pallas import tpu_sc as plsc

B = 4
N = 20736
WPB = 8
CHUNK = N // WPB
STEPS = CHUNK // 16
NPICK = 300
PPAD = 304
PBLK = PPAD // 16
TOPK = 32
NGT = 10
IOU_THR = 0.5

_f32 = jnp.float32
_i32 = jnp.int32

Y1O, X1O, Y2O, X2O, ARO = (k * CHUNK for k in range(5))


def _kernel_body(anch_hbm, delt_hbm, lab_hbm, gt_hbm, roi_hbm, gti_hbm,
                 av, dv, bd, sc, cand, stage, gtv,
                 picks, merged, gtid, roi_st, gti_st, shared):
    c = lax.axis_index("c")
    s = lax.axis_index("s")
    b = 2 * c + s // WPB
    part = s % WPB
    gbase = (s // WPB) * WPB
    obase = 8 - gbase
    is_leader = part == 0
    lane = lax.iota(_i32, 16)

    def _bcast(v, j):
        return v[jnp.full((16,), j, _i32)]

    def _hmax(v):
        for off in (8, 4, 2, 1):
            v = jnp.maximum(v, v[lane ^ off])
        return v

    def _hmin(v):
        for off in (8, 4, 2, 1):
            v = jnp.minimum(v, v[lane ^ off])
        return v

    def _lexmax(v, j, l):
        for off in (8, 4, 2, 1):
            pv = v[lane ^ off]
            pj = j[lane ^ off]
            pL = l[lane ^ off]
            take = (pv > v) | ((pv == v) & (pj < j))
            v = jnp.where(take, pv, v)
            j = jnp.where(take, pj, j)
            l = jnp.where(take, pL, l)
        return v, j, l

    for k in range(4):
        pltpu.sync_copy(
            anch_hbm.at[pl.ds((b * 4 + k) * N + part * CHUNK, CHUNK)],
            av.at[k])
        pltpu.sync_copy(
            delt_hbm.at[pl.ds((b * 4 + k) * N + part * CHUNK, CHUNK)],
            dv.at[k])
    pltpu.sync_copy(lab_hbm.at[pl.ds(b * N + part * CHUNK, CHUNK)], sc)

    @pl.when(is_leader)
    def _():
        pltpu.sync_copy(gt_hbm.at[pl.ds(b * 64, 64)], gtv)

    def dec(t):
        sl = pl.ds(t * 16, 16)
        a0 = av[0, sl]; a1 = av[1, sl]; a2 = av[2, sl]; a3 = av[3, sl]
        d0 = dv[0, sl]; d1 = dv[1, sl]; d2 = dv[2, sl]; d3 = dv[3, sl]
        aw = a3 - a1
        ah = a2 - a0
        acx = a1 + 0.5 * aw
        acy = a0 + 0.5 * ah
        bw = jnp.exp(d3) * aw
        bh = jnp.exp(d2) * ah
        bcx = d1 * aw + acx
        bcy = d0 * ah + acy
        y1 = bcy - 0.5 * bh
        x1 = bcx - 0.5 * bw
        y2 = bh + y1
        x2 = bw + x1
        bd[pl.ds(Y1O + t * 16, 16)] = y1
        bd[pl.ds(X1O + t * 16, 16)] = x1
        bd[pl.ds(Y2O + t * 16, 16)] = y2
        bd[pl.ds(X2O + t * 16, 16)] = x2
        bd[pl.ds(ARO + t * 16, 16)] = (
            jnp.maximum(y2 - y1, 0.0) * jnp.maximum(x2 - x1, 0.0))

    plsc.parallel_loop(0, STEPS, 1, unroll=4)(dec)

    ninf = jnp.full((16,), -jnp.inf, _f32)
    zero_i = jnp.zeros((16,), _i32)
    top2_0 = (ninf, zero_i, ninf, zero_i)

    def _top2_step(t, carry, s1):
        v1, j1, v2, j2 = carry
        jvv = t * 16 + lane
        u1 = s1 > v1
        u2 = (s1 > v2) & (~u1)
        nv2 = jnp.where(u1, v1, jnp.where(u2, s1, v2))
        nj2 = jnp.where(u1, j1, jnp.where(u2, jvv, j2))
        nv1 = jnp.where(u1, s1, v1)
        nj1 = jnp.where(u1, jvv, j1)
        return nv1, nj1, nv2, nj2

    def am0(t, carry):
        return _top2_step(t, carry, sc[pl.ds(t * 16, 16)])

    top2 = plsc.parallel_loop(0, STEPS, 1, unroll=4, carry=top2_0)(am0)

    def cond(carry):
        return carry[2] == 1

    def round_body(carry):
        i, r, _, v1, j1, v2, j2 = carry
        actives = i < NPICK
        activev = zero_i + i < NPICK

        c1v, c1j, _ = _lexmax(v1, j1, lane)
        winm = (v1 == c1v) & (j1 == c1j)
        c2v, c2j, _ = _lexmax(jnp.where(winm, v2, v1),
                              jnp.where(winm, j2, j1), lane)

        jboth = jnp.where(lane < 8, c1j, c2j)
        row = plsc.load_gather(
            bd, [jboth + CHUNK * jnp.clip((lane & 7) - 2, 0, 4)])
        row = jnp.where(lane == 0, c1v, row)
        row = jnp.where(lane == 1, (part * CHUNK + c1j).astype(_f32), row)
        row = jnp.where(lane == 7, i.astype(_f32), row)
        row = jnp.where(lane == 8, c2v, row)
        row = jnp.where(lane == 9, (part * CHUNK + c2j).astype(_f32), row)
        cand[...] = row
        p = r % 2
        pltpu.sync_copy(cand, shared.at[pl.ds(p * 256 + s * 16, 16)])
        plsc.subcore_barrier()
        pltpu.sync_copy(shared.at[pl.ds(p * 256, 256)], stage)

        base = gbase * 16 + (lane & 7) * 16 + (lane >> 3) * 8
        sv = plsc.load_gather(stage, [base])
        jv = plsc.load_gather(stage, [base + 1]).astype(_i32)
        y1v = plsc.load_gather(stage, [base + 2])
        x1v = plsc.load_gather(stage, [base + 3])
        y2v = plsc.load_gather(stage, [base + 4])
        x2v = plsc.load_gather(stage, [base + 5])
        avv = plsc.load_gather(stage, [base + 6])

        w1v, w1j, w1l = _lexmax(sv, jv, lane)
        w1y1 = y1v[w1l]; w1x1 = x1v[w1l]
        w1y2 = y2v[w1l]; w1x2 = x2v[w1l]
        w1a = avv[w1l]
        valid1 = (w1v > 0.0) & activev
        w1y1d = jnp.where(valid1, w1y1, 2.0)
        w1x1d = jnp.where(valid1, w1x1, 2.0)
        w1y2d = jnp.where(valid1, w1y2, -2.0)
        w1x2d = jnp.where(valid1, w1x2, -2.0)

        yy1 = jnp.maximum(w1y1d, y1v)
        xx1 = jnp.maximum(w1x1d, x1v)
        yy2 = jnp.minimum(w1y2d, y2v)
        xx2 = jnp.minimum(w1x2d, x2v)
        inter = jnp.maximum(yy2 - yy1, 0.0) * jnp.maximum(xx2 - xx1, 0.0)
        dead = (inter > IOU_THR * jnp.maximum(avv + w1a - inter, 1e-8)) \
            | (jv == w1j)

        w2v, w2j, w2l = _lexmax(jnp.where(dead, -jnp.inf, sv), jv, lane)
        w2y1 = y1v[w2l]; w2x1 = x1v[w2l]
        w2y2 = y2v[w2l]; w2x2 = x2v[w2l]
        w2a = avv[w2l]
        di = jnp.where(dead, 1, 0)
        pairdead = _hmax(di & di[lane ^ 8])
        ok2v = (pairdead == 0) & (zero_i + i + 1 < NPICK) & activev
        ok2s = jnp.where(ok2v, 1, 0)[0] == 1
        valid2 = (w2v > 0.0) & ok2v
        w2y1d = jnp.where(valid2, w2y1, 2.0)
        w2x1d = jnp.where(valid2, w2x1, 2.0)
        w2y2d = jnp.where(valid2, w2y2, -2.0)
        w2x2d = jnp.where(valid2, w2x2, -2.0)

        neg1 = jnp.full((16,), -1.0, _f32)
        own1 = (w1j >= part * CHUNK) & (w1j < (part + 1) * CHUNK) & activev
        plsc.store_scatter(sc, [jnp.clip(w1j - part * CHUNK, 0, CHUNK - 1)],
                           neg1, mask=own1 & (lane == 0))
        own2 = (w2j >= part * CHUNK) & (w2j < (part + 1) * CHUNK) & ok2v
        plsc.store_scatter(sc, [jnp.clip(w2j - part * CHUNK, 0, CHUNK - 1)],
                           neg1, mask=own2 & (lane == 0))

        @pl.when(is_leader)
        def _():
            px1 = jnp.where(lane == 2, w1y1, 0.0)
            px1 = jnp.where(lane == 3, w1x1, px1)
            px1 = jnp.where(lane == 4, w1y2, px1)
            px1 = jnp.where(lane == 5, w1x2, px1)
            px1 = jnp.where(valid1, px1, 0.0)
            pidx = PPAD * jnp.clip(lane - 2, 0, 3)
            lmask = (lane >= 2) & (lane < 6)
            plsc.store_scatter(picks, [i + pidx], px1,
                               mask=lmask & activev)
            px2 = jnp.where(lane == 2, w2y1, 0.0)
            px2 = jnp.where(lane == 3, w2x1, px2)
            px2 = jnp.where(lane == 4, w2y2, px2)
            px2 = jnp.where(lane == 5, w2x2, px2)
            px2 = jnp.where(valid2, px2, 0.0)
            plsc.store_scatter(picks, [i + 1 + pidx], px2,
                               mask=lmask & ok2v)

        def fs(t, carry):
            y1c = bd[pl.ds(Y1O + t * 16, 16)]
            x1c = bd[pl.ds(X1O + t * 16, 16)]
            y2c = bd[pl.ds(Y2O + t * 16, 16)]
            x2c = bd[pl.ds(X2O + t * 16, 16)]
            ac = bd[pl.ds(ARO + t * 16, 16)]
            s0 = sc[pl.ds(t * 16, 16)]
            i1 = (jnp.maximum(jnp.minimum(w1y2d, y2c)
                              - jnp.maximum(w1y1d, y1c), 0.0)
                  * jnp.maximum(jnp.minimum(w1x2d, x2c)
                                - jnp.maximum(w1x1d, x1c), 0.0))
            i2 = (jnp.maximum(jnp.minimum(w2y2d, y2c)
                              - jnp.maximum(w2y1d, y1c), 0.0)
                  * jnp.maximum(jnp.minimum(w2x2d, x2c)
                                - jnp.maximum(w2x1d, x1c), 0.0))
            supp = (i1 > IOU_THR * jnp.maximum(ac + w1a - i1, 1e-8)) \
                | (i2 > IOU_THR * jnp.maximum(ac + w2a - i2, 1e-8))
            s1 = jnp.where(supp, -1.0, s0)
            sc[pl.ds(t * 16, 16)] = s1
            return _top2_step(t, carry, s1)

        nt = plsc.parallel_loop(0, STEPS, 1, unroll=16, carry=top2_0)(fs)

        orow = stage[pl.ds(obase * 16, 16)]
        conts = (i < NPICK) | (orow[7] < float(NPICK))
        ni = jnp.where(actives, i + 1 + jnp.where(ok2s, 1, 0), i)
        return (ni, r + 1, jnp.where(conts, _i32(1), _i32(0)),
                nt[0], nt[1], nt[2], nt[3])

    lax.while_loop(cond, round_body,
                   (_i32(0), _i32(0), _i32(1),
                    top2[0], top2[1], top2[2], top2[3]))

    carry0 = (ninf, zero_i)

    @pl.when(is_leader)
    def _():
        gr_y1 = gtv[pl.ds(0, 16)]
        gr_x1 = gtv[pl.ds(16, 16)]
        gr_y2 = gtv[pl.ds(32, 16)]
        gr_x2 = gtv[pl.ds(48, 16)]

        def pc(tb, _):
            sl = pl.ds(tb * 16, 16)
            p0 = jnp.clip(picks[pl.ds(0 * PPAD + tb * 16, 16)], 0.0, 1.0)
            p1 = jnp.clip(picks[pl.ds(1 * PPAD + tb * 16, 16)], 0.0, 1.0)
            p2 = jnp.clip(picks[pl.ds(2 * PPAD + tb * 16, 16)], 0.0, 1.0)
            p3 = jnp.clip(picks[pl.ds(3 * PPAD + tb * 16, 16)], 0.0, 1.0)
            pa = jnp.maximum(p2 - p0, 0.0) * jnp.maximum(p3 - p1, 0.0)
            mg = jnp.full((16,), -1.0, _f32)
            gi = jnp.zeros((16,), _i32)
            for g in range(NGT):
                gy1 = _bcast(gr_y1, g); gx1 = _bcast(gr_x1, g)
                gy2 = _bcast(gr_y2, g); gx2 = _bcast(gr_x2, g)
                gar = (jnp.maximum(gy2 - gy1, 0.0)
                       * jnp.maximum(gx2 - gx1, 0.0))
                yy1 = jnp.maximum(p0, gy1)
                xx1 = jnp.maximum(p1, gx1)
                yy2 = jnp.minimum(p2, gy2)
                xx2 = jnp.minimum(p3, gx2)
                inter = (jnp.maximum(yy2 - yy1, 0.0)
                         * jnp.maximum(xx2 - xx1, 0.0))
                iou = inter / jnp.maximum(pa + gar - inter, 1e-8)
                upd = iou > mg
                mg = jnp.where(upd, iou, mg)
                gi = jnp.where(upd, _i32(g), gi)
            jv = tb * 16 + lane
            mg = jnp.where(jv < NPICK, mg, -2.0)
            merged[sl] = mg
            gtid[sl] = gi
            return 0

        lax.fori_loop(0, PBLK, pc, 0)

        def ext(k, _):
            def am(t, carry):
                mv, mj = carry
                v = merged[pl.ds(t * 16, 16)]
                jv = t * 16 + lane
                upd = v > mv
                return jnp.where(upd, v, mv), jnp.where(upd, jv, mj)

            mv, mj = lax.fori_loop(0, PBLK, am, carry0)
            m = _hmax(mv)
            jsel = _hmin(jnp.where(mv == m, mj, _i32(1 << 30)))
            coords = plsc.load_gather(
                picks, [jsel + PPAD * jnp.minimum(lane, 3)])
            coords = jnp.clip(coords, 0.0, 1.0)
            plsc.store_scatter(roi_st, [4 * k + lane], coords, mask=lane < 4)
            gsel = plsc.load_gather(gtid, [jsel])
            plsc.store_scatter(gti_st, [jnp.zeros((16,), _i32) + k], gsel,
                               mask=lane == 0)
            plsc.store_scatter(merged, [jsel],
                               jnp.full((16,), -3.0, _f32), mask=lane == 0)
            return 0

        lax.fori_loop(0, TOPK, ext, 0)
        pltpu.sync_copy(roi_st, roi_hbm.at[pl.ds(b * TOPK * 4, TOPK * 4)])
        pltpu.sync_copy(gti_st, gti_hbm.at[pl.ds(b * TOPK, TOPK)])


_nms_call = pl.kernel(
    _kernel_body,
    out_type=(jax.ShapeDtypeStruct((B * TOPK * 4,), _f32),
              jax.ShapeDtypeStruct((B * TOPK,), _i32)),
    mesh=plsc.VectorSubcoreMesh(core_axis_name="c", subcore_axis_name="s"),
    compiler_params=pltpu.CompilerParams(needs_layout_passes=False,
                                         use_tc_tiling_on_sc=False),
    scratch_types=[
        pltpu.VMEM((4, CHUNK), _f32),
        pltpu.VMEM((4, CHUNK), _f32),
        pltpu.VMEM((5 * CHUNK,), _f32),
        pltpu.VMEM((CHUNK,), _f32),
        pltpu.VMEM((16,), _f32),
        pltpu.VMEM((256,), _f32),
        pltpu.VMEM((64,), _f32),
        pltpu.VMEM((4 * PPAD,), _f32),
        pltpu.VMEM((PPAD,), _f32),
        pltpu.VMEM((PPAD,), _i32),
        pltpu.VMEM((TOPK * 4,), _f32),
        pltpu.VMEM((TOPK,), _i32),
        pltpu.VMEM_SHARED((2 * 16 * 16,), _f32),
    ],
)


@jax.jit
def kernel(rpn_bbox_deltas, rpn_labels, anchors, gt_boxes):
    anch_t = anchors.transpose(0, 2, 1).reshape(-1)
    delt_t = rpn_bbox_deltas.reshape(B, N, 4).transpose(0, 2, 1).reshape(-1)
    lab = rpn_labels.reshape(-1)
    gt_t = jnp.pad(gt_boxes.transpose(0, 2, 1),
                   ((0, 0), (0, 0), (0, 16 - NGT))).reshape(-1)
    roi_pos, gt_idx = _nms_call(anch_t, delt_t, lab, gt_t)
    roi = jnp.concatenate(
        [roi_pos.reshape(B, TOPK, 4), jnp.zeros((B, 128 - TOPK, 4), _f32)],
        axis=1)
    return lax.stop_gradient(roi), lax.stop_gradient(gt_idx.reshape(B, TOPK))
